# baseline (speedup 1.0000x reference)
import functools

import jax
import jax.numpy as jnp
from jax import lax
from jax.experimental import pallas as pl
from jax.experimental.pallas import tpu as pltpu
from jax.experimental.pallas import tpu_sc as plsc

_BETA = 0.25
_K = 8192
_D = 256
_M = 16384
_MBLK = 256
_NBLK = 256
_NSTEPS = _K // _NBLK


def _vq_argmin_body(emb_ref, zf_ref, idx_ref, mind_ref, argd_ref):
    n = pl.program_id(1)
    zf = zf_ref[...]
    e_t = emb_ref[n]
    c = lax.dot_general(zf, e_t, (((1,), (0,)), ((), ())),
                        preferred_element_type=jnp.float32)
    a = jnp.sum(zf * zf, axis=1, keepdims=True)
    b = jnp.sum(e_t * e_t, axis=0, keepdims=True)
    d = a + b - 2.0 * c
    loc_min = jnp.min(d, axis=1, keepdims=True)
    lanes = lax.broadcasted_iota(jnp.int32, (_MBLK, _NBLK), 1)
    cand = jnp.where(d == loc_min, lanes, jnp.int32(2 ** 30))
    loc_arg = jnp.min(cand, axis=1, keepdims=True) + n * _NBLK

    @pl.when(n == 0)
    def _():
        mind_ref[...] = loc_min
        argd_ref[...] = loc_arg

    @pl.when(n > 0)
    def _():
        prev = mind_ref[...]
        upd = loc_min < prev
        argd_ref[...] = jnp.where(upd, loc_arg, argd_ref[...])
        mind_ref[...] = jnp.where(upd, loc_min, prev)

    @pl.when(n == _NSTEPS - 1)
    def _():
        idx_ref[...] = argd_ref[...]


def _argmin_indices(zf, emb_blocks):
    return pl.pallas_call(
        _vq_argmin_body,
        grid=(_M // _MBLK, _NSTEPS),
        in_specs=[
            pl.BlockSpec((_NSTEPS, _D, _NBLK), lambda m, n: (0, 0, 0)),
            pl.BlockSpec((_MBLK, _D), lambda m, n: (m, 0)),
        ],
        out_specs=pl.BlockSpec((_MBLK, 1), lambda m, n: (m, 0)),
        out_shape=jax.ShapeDtypeStruct((_M, 1), jnp.int32),
        scratch_shapes=[
            pltpu.VMEM((_MBLK, 1), jnp.float32),
            pltpu.VMEM((_MBLK, 1), jnp.int32),
        ],
    )(emb_blocks, zf)


def _gather_rows_sc(emb, idx):
    mesh = plsc.VectorSubcoreMesh(core_axis_name="c", subcore_axis_name="s")

    @functools.partial(
        pl.kernel,
        mesh=mesh,
        out_type=jax.ShapeDtypeStruct((_M, _D), jnp.float32),
        scratch_types=[
            pltpu.VMEM((128,), jnp.int32),
            pltpu.VMEM((128, _D), jnp.float32),
            pltpu.SemaphoreType.DMA,
        ],
    )
    def gk(emb_hbm, idx_hbm, out_hbm, idx_v, rows_v, sem):
        wid = lax.axis_index("s") * 2 + lax.axis_index("c")
        base = wid * 512
        for j in range(4):
            off = base + j * 128
            pltpu.sync_copy(idx_hbm.at[pl.ds(off, 128)], idx_v)
            pltpu.async_copy(emb_hbm.at[idx_v], rows_v, sem).wait()
            pltpu.sync_copy(rows_v, out_hbm.at[pl.ds(off, 128)])

    return gk(emb, idx)


def _st_loss_body(z_ref, zq_ref, out_ref, loss_ref, acc_ref):
    b = pl.program_id(0)
    j = pl.program_id(1)
    zq = zq_ref[0, 0]
    zb = z_ref[0]
    eye = (lax.broadcasted_iota(jnp.int32, (128, 128), 0)
           == lax.broadcasted_iota(jnp.int32, (128, 128), 1)
           ).astype(jnp.float32)
    zq_t = lax.dot_general(zq, eye, (((0,), (0,)), ((), ())),
                           preferred_element_type=jnp.float32,
                           precision=lax.Precision.HIGHEST)
    diff = zq_t - zb
    out_ref[0] = zb + diff
    s = jnp.sum(diff * diff)

    @pl.when((b == 0) & (j == 0))
    def _():
        acc_ref[0, 0] = s

    @pl.when((b > 0) | (j > 0))
    def _():
        acc_ref[0, 0] = acc_ref[0, 0] + s

    @pl.when((b == 15) & (j == 7))
    def _():
        loss_ref[0, 0] = acc_ref[0, 0]


def _st_and_loss(z3, zq4):
    return pl.pallas_call(
        _st_loss_body,
        grid=(16, 8),
        in_specs=[
            pl.BlockSpec((1, _D, 128), lambda b, j: (b, 0, j)),
            pl.BlockSpec((1, 1, 128, _D), lambda b, j: (b, j, 0, 0)),
        ],
        out_specs=[
            pl.BlockSpec((1, _D, 128), lambda b, j: (b, 0, j)),
            pl.BlockSpec(memory_space=pltpu.SMEM),
        ],
        out_shape=[
            jax.ShapeDtypeStruct((16, _D, 1024), jnp.float32),
            jax.ShapeDtypeStruct((1, 1), jnp.float32),
        ],
        scratch_shapes=[pltpu.SMEM((1, 1), jnp.float32)],
    )(z3, zq4)


def kernel(z, emb_weight):
    zf = jnp.transpose(z, (0, 2, 3, 1)).reshape(_M, _D)
    emb_blocks = emb_weight.reshape(_NSTEPS, _NBLK, _D).transpose(0, 2, 1)

    idx2 = _argmin_indices(zf, emb_blocks)
    idx = idx2.reshape(_M)

    zq = _gather_rows_sc(emb_weight, idx)

    z3 = z.reshape(16, _D, 1024)
    zq4 = zq.reshape(16, 8, 128, _D)
    out3, loss_sum = _st_and_loss(z3, zq4)

    mean = loss_sum[0, 0] / jnp.float32(_M * _D)
    loss = _BETA * mean + mean
    z_q_out = out3.reshape(16, _D, 32, 32)
    return z_q_out, loss, idx


# speedup vs baseline: 1.5563x; 1.5563x over previous
import functools

import jax
import jax.numpy as jnp
from jax import lax
from jax.experimental import pallas as pl
from jax.experimental.pallas import tpu as pltpu
from jax.experimental.pallas import tpu_sc as plsc

_BETA = 0.25
_K = 8192
_D = 256
_M = 16384
_MBLK = 256
_NBLK = 512
_NSTEPS = _K // _NBLK


def _vq_argmin_body(emb_ref, zf_ref, idx_ref, mind_ref, argd_ref):
    n = pl.program_id(1)
    zf = zf_ref[...]
    e_t = emb_ref[n]
    c = lax.dot_general(zf, e_t, (((1,), (0,)), ((), ())),
                        preferred_element_type=jnp.float32)
    ef = e_t.astype(jnp.float32)
    b = jnp.sum(ef * ef, axis=0, keepdims=True)
    d = b - 2.0 * c
    loc_min = jnp.min(d, axis=1, keepdims=True)
    lanes = lax.broadcasted_iota(jnp.int32, (_MBLK, _NBLK), 1)
    cand = jnp.where(d == loc_min, lanes, jnp.int32(2 ** 30))
    loc_arg = jnp.min(cand, axis=1, keepdims=True) + n * _NBLK

    @pl.when(n == 0)
    def _():
        mind_ref[...] = loc_min
        argd_ref[...] = loc_arg

    @pl.when(n > 0)
    def _():
        prev = mind_ref[...]
        upd = loc_min < prev
        argd_ref[...] = jnp.where(upd, loc_arg, argd_ref[...])
        mind_ref[...] = jnp.where(upd, loc_min, prev)

    @pl.when(n == _NSTEPS - 1)
    def _():
        idx_ref[...] = argd_ref[...]


def _argmin_indices(zf, emb_blocks):
    return pl.pallas_call(
        _vq_argmin_body,
        grid=(_M // _MBLK, _NSTEPS),
        in_specs=[
            pl.BlockSpec((_NSTEPS, _D, _NBLK), lambda m, n: (0, 0, 0)),
            pl.BlockSpec((_MBLK, _D), lambda m, n: (m, 0)),
        ],
        out_specs=pl.BlockSpec((_MBLK, 1), lambda m, n: (m, 0)),
        out_shape=jax.ShapeDtypeStruct((_M, 1), jnp.int32),
        scratch_shapes=[
            pltpu.VMEM((_MBLK, 1), jnp.float32),
            pltpu.VMEM((_MBLK, 1), jnp.int32),
        ],
    )(emb_blocks, zf)


def _gather_rows_sc(emb, idx):
    mesh = plsc.VectorSubcoreMesh(core_axis_name="c", subcore_axis_name="s")

    @functools.partial(
        pl.kernel,
        mesh=mesh,
        out_type=jax.ShapeDtypeStruct((_M, _D), jnp.float32),
        scratch_types=[
            pltpu.VMEM((128,), jnp.int32),
            pltpu.VMEM((128, _D), jnp.float32),
            pltpu.SemaphoreType.DMA,
        ],
    )
    def gk(emb_hbm, idx_hbm, out_hbm, idx_v, rows_v, sem):
        wid = lax.axis_index("s") * 2 + lax.axis_index("c")
        base = wid * 512
        for j in range(4):
            off = base + j * 128
            pltpu.sync_copy(idx_hbm.at[pl.ds(off, 128)], idx_v)
            pltpu.async_copy(emb_hbm.at[idx_v], rows_v, sem).wait()
            pltpu.sync_copy(rows_v, out_hbm.at[pl.ds(off, 128)])

    return gk(emb, idx)


def _st_loss_body(z_ref, zq_ref, out_ref, loss_ref, acc_ref):
    b = pl.program_id(0)
    j = pl.program_id(1)
    zq = zq_ref[0, 0]
    zb = z_ref[0]
    eye = (lax.broadcasted_iota(jnp.int32, (128, 128), 0)
           == lax.broadcasted_iota(jnp.int32, (128, 128), 1)
           ).astype(jnp.float32)
    zq_t = lax.dot_general(zq, eye, (((0,), (0,)), ((), ())),
                           preferred_element_type=jnp.float32,
                           precision=lax.Precision.HIGHEST)
    diff = zq_t - zb
    out_ref[0] = zb + diff
    s = jnp.sum(diff * diff)

    @pl.when((b == 0) & (j == 0))
    def _():
        acc_ref[0, 0] = s

    @pl.when((b > 0) | (j > 0))
    def _():
        acc_ref[0, 0] = acc_ref[0, 0] + s

    @pl.when((b == 15) & (j == 7))
    def _():
        loss_ref[0, 0] = acc_ref[0, 0]


def _st_and_loss(z3, zq4):
    return pl.pallas_call(
        _st_loss_body,
        grid=(16, 8),
        in_specs=[
            pl.BlockSpec((1, _D, 128), lambda b, j: (b, 0, j)),
            pl.BlockSpec((1, 1, 128, _D), lambda b, j: (b, j, 0, 0)),
        ],
        out_specs=[
            pl.BlockSpec((1, _D, 128), lambda b, j: (b, 0, j)),
            pl.BlockSpec(memory_space=pltpu.SMEM),
        ],
        out_shape=[
            jax.ShapeDtypeStruct((16, _D, 1024), jnp.float32),
            jax.ShapeDtypeStruct((1, 1), jnp.float32),
        ],
        scratch_shapes=[pltpu.SMEM((1, 1), jnp.float32)],
    )(z3, zq4)


def kernel(z, emb_weight):
    zf = jnp.transpose(z, (0, 2, 3, 1)).reshape(_M, _D).astype(jnp.bfloat16)
    emb_blocks = (emb_weight.reshape(_NSTEPS, _NBLK, _D)
                  .transpose(0, 2, 1).astype(jnp.bfloat16))

    idx2 = _argmin_indices(zf, emb_blocks)
    idx = idx2.reshape(_M)

    zq = _gather_rows_sc(emb_weight, idx)

    z3 = z.reshape(16, _D, 1024)
    zq4 = zq.reshape(16, 8, 128, _D)
    out3, loss_sum = _st_and_loss(z3, zq4)

    mean = loss_sum[0, 0] / jnp.float32(_M * _D)
    loss = _BETA * mean + mean
    z_q_out = out3.reshape(16, _D, 32, 32)
    return z_q_out, loss, idx
